# baseline (speedup 1.0000x reference)
import jax
import jax.numpy as jnp
from jax import lax
from jax.experimental import pallas as pl
from jax.experimental.pallas import tpu as pltpu
from jax.experimental.pallas import tpu_sc as plsc

_N_EMBED = 8192
_E_DIM = 256
_M_TOTAL = 16384
_BM = 256
_BMC = 2048
_WINDOW = 2736
_SC_CHUNK = 128


def _prep_body(e_ref, ehi_ref, esq_ref):
    e = e_ref[...]
    ehi_ref[...] = e.astype(jnp.bfloat16)
    esq_ref[...] = jnp.sum(e * e, axis=1)[None, :]


def _argmax_body(z_ref, ehi_ref, esq_ref, idx_ref):
    z = z_ref[...]
    zb = z.astype(jnp.bfloat16)
    zsq = jnp.sum(z * z, axis=1, keepdims=True)
    big = jnp.int32(2**30)
    acc_v = None
    for lo in range(0, _N_EMBED, _WINDOW):
        hi = min(lo + _WINDOW, _N_EMBED)
        dot = jax.lax.dot_general(zb, ehi_ref[lo:hi, :],
                                  (((1,), (1,)), ((), ())),
                                  preferred_element_type=jnp.float32)
        sim = (zsq + esq_ref[:, lo:hi]) - 2.0 * dot
        iota = jax.lax.broadcasted_iota(jnp.int32, (1, hi - lo), 1) + lo
        wv = jnp.max(sim, axis=1, keepdims=True)
        wi = jnp.min(jnp.where(sim == wv, iota, big),
                     axis=1, keepdims=True)
        if acc_v is None:
            acc_v, acc_i = wv, wi
        else:
            keep = acc_v > wv
            tie = (acc_v == wv) & (acc_i < wi)
            acc_i = jnp.where(keep | tie, acc_i, wi)
            acc_v = jnp.where(keep, acc_v, wv)
        acc_v = acc_v.astype(jnp.bfloat16).astype(jnp.float32)

    idx_ref[0, 0, :] = acc_i[:, 0]


def _st_body(z_ref, zq_ref, out_ref, rss_ref):
    z = z_ref[...]
    d = zq_ref[...] - z
    out_ref[...] = z + d
    rss_ref[0, 0, :] = jnp.sum(d * d, axis=1)


def _sc_gather(idx_hbm, table_hbm, out_hbm, idx_v, rows0, rows1, sem0, sem1):
    nc = 2
    wid = lax.axis_index("s") * nc + lax.axis_index("c")
    rows_per_w = _M_TOTAL // 32
    base = wid * rows_per_w
    pltpu.sync_copy(idx_hbm.at[pl.ds(base, rows_per_w)], idx_v)
    bufs = (rows0, rows1)
    sems = (sem0, sem1)
    n_chunks = rows_per_w // _SC_CHUNK
    copies = [None, None]
    for c in range(n_chunks):
        b = c % 2
        if copies[b] is not None:
            copies[b].wait()
            pltpu.sync_copy(bufs[b],
                            out_hbm.at[pl.ds(base + (c - 2) * _SC_CHUNK,
                                             _SC_CHUNK)])
        copies[b] = pltpu.async_copy(
            table_hbm.at[idx_v.at[pl.ds(c * _SC_CHUNK, _SC_CHUNK)]],
            bufs[b], sems[b])
    for c in range(n_chunks - 2, n_chunks):
        b = c % 2
        copies[b].wait()
        pltpu.sync_copy(bufs[b], out_hbm.at[pl.ds(base + c * _SC_CHUNK,
                                                  _SC_CHUNK)])


def kernel(z, embedding_weight):
    z2 = z.reshape(_M_TOTAL, _E_DIM)
    ehi, esq = pl.pallas_call(
        _prep_body,
        out_shape=[
            jax.ShapeDtypeStruct((_N_EMBED, _E_DIM), jnp.bfloat16),
            jax.ShapeDtypeStruct((1, _N_EMBED), jnp.float32),
        ],
    )(embedding_weight)

    n_blocks = _M_TOTAL // _BM
    idx = pl.pallas_call(
        _argmax_body,
        grid=(n_blocks,),
        in_specs=[
            pl.BlockSpec((_BM, _E_DIM), lambda i: (i, 0)),
            pl.BlockSpec((_N_EMBED, _E_DIM), lambda i: (0, 0)),
            pl.BlockSpec((1, _N_EMBED), lambda i: (0, 0)),
        ],
        out_specs=pl.BlockSpec((1, 1, _BM), lambda i: (i, 0, 0)),
        out_shape=jax.ShapeDtypeStruct((n_blocks, 1, _BM), jnp.int32),
        compiler_params=pltpu.CompilerParams(
            dimension_semantics=("parallel",)),
    )(z2, ehi, esq)
    idx_flat = idx.reshape(_M_TOTAL)

    sc_gather = pl.kernel(
        _sc_gather,
        mesh=plsc.VectorSubcoreMesh(core_axis_name="c", subcore_axis_name="s"),
        out_type=jax.ShapeDtypeStruct((_M_TOTAL, _E_DIM), jnp.float32),
        scratch_types=[
            pltpu.VMEM((_M_TOTAL // 32,), jnp.int32),
            pltpu.VMEM((_SC_CHUNK, _E_DIM), jnp.float32),
            pltpu.VMEM((_SC_CHUNK, _E_DIM), jnp.float32),
            pltpu.SemaphoreType.DMA,
            pltpu.SemaphoreType.DMA,
        ],
    )
    zq = sc_gather(idx_flat, embedding_weight)

    nc_blocks = _M_TOTAL // _BMC
    out, rss = pl.pallas_call(
        _st_body,
        grid=(nc_blocks,),
        in_specs=[
            pl.BlockSpec((_BMC, _E_DIM), lambda i: (i, 0)),
            pl.BlockSpec((_BMC, _E_DIM), lambda i: (i, 0)),
        ],
        out_specs=[
            pl.BlockSpec((_BMC, _E_DIM), lambda i: (i, 0)),
            pl.BlockSpec((1, 1, _BMC), lambda i: (i, 0, 0)),
        ],
        out_shape=[
            jax.ShapeDtypeStruct((_M_TOTAL, _E_DIM), jnp.float32),
            jax.ShapeDtypeStruct((nc_blocks, 1, _BMC), jnp.float32),
        ],
        compiler_params=pltpu.CompilerParams(
            dimension_semantics=("parallel",)),
    )(z2, zq)

    total = jnp.sum(rss)
    vq_loss = total * jnp.float32(1.0 / (_M_TOTAL * _E_DIM))
    commitment_loss = total * jnp.float32(0.25 / (_M_TOTAL * _E_DIM))
    return (out.reshape(z.shape), vq_loss, commitment_loss, idx_flat)


# speedup vs baseline: 1.0133x; 1.0133x over previous
import jax
import jax.numpy as jnp
from jax import lax
from jax.experimental import pallas as pl
from jax.experimental.pallas import tpu as pltpu
from jax.experimental.pallas import tpu_sc as plsc

_N_EMBED = 8192
_E_DIM = 256
_M_TOTAL = 16384
_BM = 256
_BMC = 2048
_WINDOW = 2736
_SC_CHUNK = 128


def _prep_body(e_ref, ehi_ref, esq_ref):
    e = e_ref[...]
    ehi_ref[...] = e.astype(jnp.bfloat16)
    esq_ref[...] = jnp.sum(e * e, axis=1)[None, :]


def _argmax_body(z_ref, ehi_ref, esq_ref, idx_ref):
    z = z_ref[...]
    zm = (z * jnp.float32(-2.0)).astype(jnp.bfloat16)
    zsq = jnp.sum(z * z, axis=1, keepdims=True)
    big = jnp.int32(2**30)
    acc_v = None
    for lo in range(0, _N_EMBED, _WINDOW):
        hi = min(lo + _WINDOW, _N_EMBED)
        ndot2 = jax.lax.dot_general(zm, ehi_ref[lo:hi, :],
                                    (((1,), (1,)), ((), ())),
                                    preferred_element_type=jnp.float32)
        sim = (zsq + esq_ref[:, lo:hi]) + ndot2
        iota = jax.lax.broadcasted_iota(jnp.int32, (1, hi - lo), 1) + lo
        wv = jnp.max(sim, axis=1, keepdims=True)
        wi = jnp.min(jnp.where(sim == wv, iota, big),
                     axis=1, keepdims=True)
        if acc_v is None:
            acc_v, acc_i = wv, wi
        else:
            keep = acc_v > wv
            tie = (acc_v == wv) & (acc_i < wi)
            acc_i = jnp.where(keep | tie, acc_i, wi)
            acc_v = jnp.where(keep, acc_v, wv)
        acc_v = acc_v.astype(jnp.bfloat16).astype(jnp.float32)

    idx_ref[0, 0, :] = acc_i[:, 0]


def _st_body(z_ref, zq_ref, out_ref, rss_ref):
    z = z_ref[...]
    d = zq_ref[...] - z
    out_ref[...] = z + d
    rss_ref[0, 0, :] = jnp.sum(d * d, axis=1)


def _sc_gather(idx_hbm, table_hbm, out_hbm, idx_v, rows0, rows1, sem0, sem1):
    nc = 2
    wid = lax.axis_index("s") * nc + lax.axis_index("c")
    rows_per_w = _M_TOTAL // 32
    base = wid * rows_per_w
    pltpu.sync_copy(idx_hbm.at[pl.ds(base, rows_per_w)], idx_v)
    bufs = (rows0, rows1)
    sems = (sem0, sem1)
    n_chunks = rows_per_w // _SC_CHUNK
    copies = [None, None]
    for c in range(n_chunks):
        b = c % 2
        if copies[b] is not None:
            copies[b].wait()
            pltpu.sync_copy(bufs[b],
                            out_hbm.at[pl.ds(base + (c - 2) * _SC_CHUNK,
                                             _SC_CHUNK)])
        copies[b] = pltpu.async_copy(
            table_hbm.at[idx_v.at[pl.ds(c * _SC_CHUNK, _SC_CHUNK)]],
            bufs[b], sems[b])
    for c in range(n_chunks - 2, n_chunks):
        b = c % 2
        copies[b].wait()
        pltpu.sync_copy(bufs[b], out_hbm.at[pl.ds(base + c * _SC_CHUNK,
                                                  _SC_CHUNK)])


def kernel(z, embedding_weight):
    z2 = z.reshape(_M_TOTAL, _E_DIM)
    ehi, esq = pl.pallas_call(
        _prep_body,
        out_shape=[
            jax.ShapeDtypeStruct((_N_EMBED, _E_DIM), jnp.bfloat16),
            jax.ShapeDtypeStruct((1, _N_EMBED), jnp.float32),
        ],
    )(embedding_weight)

    n_blocks = _M_TOTAL // _BM
    idx = pl.pallas_call(
        _argmax_body,
        grid=(n_blocks,),
        in_specs=[
            pl.BlockSpec((_BM, _E_DIM), lambda i: (i, 0)),
            pl.BlockSpec((_N_EMBED, _E_DIM), lambda i: (0, 0)),
            pl.BlockSpec((1, _N_EMBED), lambda i: (0, 0)),
        ],
        out_specs=pl.BlockSpec((1, 1, _BM), lambda i: (i, 0, 0)),
        out_shape=jax.ShapeDtypeStruct((n_blocks, 1, _BM), jnp.int32),
        compiler_params=pltpu.CompilerParams(
            dimension_semantics=("parallel",)),
    )(z2, ehi, esq)
    idx_flat = idx.reshape(_M_TOTAL)

    sc_gather = pl.kernel(
        _sc_gather,
        mesh=plsc.VectorSubcoreMesh(core_axis_name="c", subcore_axis_name="s"),
        out_type=jax.ShapeDtypeStruct((_M_TOTAL, _E_DIM), jnp.float32),
        scratch_types=[
            pltpu.VMEM((_M_TOTAL // 32,), jnp.int32),
            pltpu.VMEM((_SC_CHUNK, _E_DIM), jnp.float32),
            pltpu.VMEM((_SC_CHUNK, _E_DIM), jnp.float32),
            pltpu.SemaphoreType.DMA,
            pltpu.SemaphoreType.DMA,
        ],
    )
    zq = sc_gather(idx_flat, embedding_weight)

    nc_blocks = _M_TOTAL // _BMC
    out, rss = pl.pallas_call(
        _st_body,
        grid=(nc_blocks,),
        in_specs=[
            pl.BlockSpec((_BMC, _E_DIM), lambda i: (i, 0)),
            pl.BlockSpec((_BMC, _E_DIM), lambda i: (i, 0)),
        ],
        out_specs=[
            pl.BlockSpec((_BMC, _E_DIM), lambda i: (i, 0)),
            pl.BlockSpec((1, 1, _BMC), lambda i: (i, 0, 0)),
        ],
        out_shape=[
            jax.ShapeDtypeStruct((_M_TOTAL, _E_DIM), jnp.float32),
            jax.ShapeDtypeStruct((nc_blocks, 1, _BMC), jnp.float32),
        ],
        compiler_params=pltpu.CompilerParams(
            dimension_semantics=("parallel",)),
    )(z2, zq)

    total = jnp.sum(rss)
    vq_loss = total * jnp.float32(1.0 / (_M_TOTAL * _E_DIM))
    commitment_loss = total * jnp.float32(0.25 / (_M_TOTAL * _E_DIM))
    return (out.reshape(z.shape), vq_loss, commitment_loss, idx_flat)
